# baseline (speedup 1.0000x reference)
import numpy as np
import jax
import jax.numpy as jnp
from jax.experimental import pallas as pl
from jax.experimental.pallas import tpu as pltpu

_B = 32
_V = 1_000_000
_NS = 10
_CHUNK = 8192

_KD = np.asarray(jax.random.key_data(jax.random.fold_in(jax.random.key(0), 1)))
_K1 = np.uint32(_KD[0])
_K2 = np.uint32(_KD[1])
_K3 = np.uint32(_K1 ^ _K2 ^ np.uint32(0x1BD11BDA))

_ROT_A = (13, 15, 26, 6)
_ROT_B = (17, 29, 16, 24)
_TINY = np.float32(np.finfo(np.float32).tiny)


def _threefry_bits(x1):
    ks = (jnp.uint32(_K1), jnp.uint32(_K2), jnp.uint32(_K3))
    x0 = jnp.full_like(x1, _K1)
    x1 = x1 + jnp.uint32(_K2)

    def rnd(a, b, r):
        a = a + b
        b = ((b << jnp.uint32(r)) | (b >> jnp.uint32(32 - r))) ^ a
        return a, b

    rots = (_ROT_A, _ROT_B, _ROT_A, _ROT_B, _ROT_A)
    for i in range(5):
        for r in rots[i]:
            x0, x1 = rnd(x0, x1, r)
        x0 = x0 + ks[(i + 1) % 3]
        x1 = x1 + ks[(i + 2) % 3] + jnp.uint32(i + 1)
    return x0 ^ x1


def _make(B, V, K, NS, interpret=False):
    nchunk = (V + K - 1) // K

    def body(l_ref, out_ref, m_ref, s_ref, bval_ref, bidx_ref):
        j = pl.program_id(0)

        @pl.when(j == 0)
        def _init():
            m_ref[...] = jnp.full_like(m_ref, -jnp.inf)
            s_ref[...] = jnp.zeros_like(s_ref)
            bval_ref[...] = jnp.full_like(bval_ref, -jnp.inf)
            bidx_ref[...] = jnp.zeros_like(bidx_ref)

        col = jax.lax.broadcasted_iota(jnp.int32, (B, K), 1) + j * K
        row = jax.lax.broadcasted_iota(jnp.int32, (B, K), 0)
        valid = col < V
        l = jnp.where(valid, l_ref[...], jnp.float32(-jnp.inf))

        m_old = m_ref[:, 0:1]
        m_new = jnp.maximum(m_old, jnp.max(l, axis=1, keepdims=True))
        e_sum = jnp.sum(jnp.exp(l - m_new), axis=1, keepdims=True)
        s_ref[:, 0:1] = s_ref[:, 0:1] * jnp.exp(m_old - m_new) + e_sum
        m_ref[:, 0:1] = m_new

        off = (row * V + col).astype(jnp.uint32)
        for s in range(NS):
            bits = _threefry_bits(off + jnp.uint32(s * B * V))
            fb = (bits >> jnp.uint32(9)) | jnp.uint32(0x3F800000)
            fl = jax.lax.bitcast_convert_type(fb, jnp.float32) - jnp.float32(1.0)
            u = jnp.maximum(_TINY, fl * (jnp.float32(1.0) - _TINY) + _TINY)
            g = -jnp.log(-jnp.log(u))
            phi = g + l
            vmax = jnp.max(phi, axis=1, keepdims=True)
            cand = jnp.where(phi == vmax, col, jnp.int32(0x7FFFFFFF))
            imin = jnp.min(cand, axis=1, keepdims=True)
            better = vmax > bval_ref[:, s : s + 1]
            bval_ref[:, s : s + 1] = jnp.where(better, vmax, bval_ref[:, s : s + 1])
            bidx_ref[:, s : s + 1] = jnp.where(better, imin, bidx_ref[:, s : s + 1])

        @pl.when(j == nchunk - 1)
        def _fin():
            top_prob = jnp.float32(1.0) / s_ref[:, 0:1]
            idxs = [bidx_ref[:, t : t + 1] for t in range(NS)]
            agree_f = jnp.ones_like(top_prob)
            for t in range(1, NS):
                agree_f = agree_f + (idxs[t] == idxs[0]).astype(jnp.float32)
            agree_f = agree_f * jnp.float32(1.0 / NS)
            uniq = jnp.ones_like(top_prob)
            for t in range(1, NS):
                seen = idxs[t] == idxs[0]
                for t2 in range(1, t):
                    seen = jnp.logical_or(seen, idxs[t] == idxs[t2])
                uniq = uniq + jnp.float32(1.0) - seen.astype(jnp.float32)
            out_ref[...] = jnp.zeros_like(out_ref)
            out_ref[:, 0:1] = agree_f
            out_ref[:, 1:2] = uniq * jnp.float32(1.0 / NS)
            out_ref[:, 2:3] = agree_f - top_prob

    return pl.pallas_call(
        body,
        grid=(nchunk,),
        in_specs=[pl.BlockSpec((B, K), lambda j: (0, j))],
        out_specs=pl.BlockSpec((B, 128), lambda j: (0, 0)),
        out_shape=jax.ShapeDtypeStruct((B, 128), jnp.float32),
        scratch_shapes=[
            pltpu.VMEM((B, 128), jnp.float32),
            pltpu.VMEM((B, 128), jnp.float32),
            pltpu.VMEM((B, 128), jnp.float32),
            pltpu.VMEM((B, 128), jnp.int32),
        ],
        interpret=interpret,
    )


def kernel(logits):
    out = _make(_B, _V, _CHUNK, _NS)(logits)
    return out[:, :3]


# speedup vs baseline: 1.6086x; 1.6086x over previous
import numpy as np
import jax
import jax.numpy as jnp
from jax.experimental import pallas as pl
from jax.experimental.pallas import tpu as pltpu

_B = 32
_V = 1_000_000
_NS = 10
_CHUNK = 8192

_ROT_A = (13, 15, 26, 6)
_ROT_B = (17, 29, 16, 24)
_TINY = np.float32(np.finfo(np.float32).tiny)


def _np_threefry(k1, k2, x0, x1):
    ks = [np.uint32(k1), np.uint32(k2),
          np.uint32(k1) ^ np.uint32(k2) ^ np.uint32(0x1BD11BDA)]
    x0 = np.uint32(x0 + ks[0])
    x1 = np.uint32(x1 + ks[1])
    rots = (_ROT_A, _ROT_B, _ROT_A, _ROT_B, _ROT_A)
    for i in range(5):
        for r in rots[i]:
            x0 = np.uint32(x0 + x1)
            x1 = np.uint32(((x1 << np.uint32(r)) | (x1 >> np.uint32(32 - r))) & 0xFFFFFFFF)
            x1 = np.uint32(x1 ^ x0)
        x0 = np.uint32(x0 + ks[(i + 1) % 3])
        x1 = np.uint32(x1 + ks[(i + 2) % 3] + np.uint32(i + 1))
    return x0, x1


_K1, _K2 = _np_threefry(np.uint32(0), np.uint32(0), np.uint32(0), np.uint32(1))
_K3 = np.uint32(_K1 ^ _K2 ^ np.uint32(0x1BD11BDA))


def _threefry_bits(x1):
    ks = (jnp.uint32(_K1), jnp.uint32(_K2), jnp.uint32(_K3))
    x0 = jnp.full_like(x1, _K1)
    x1 = x1 + jnp.uint32(_K2)

    def rnd(a, b, r):
        a = a + b
        b = ((b << jnp.uint32(r)) | (b >> jnp.uint32(32 - r))) ^ a
        return a, b

    rots = (_ROT_A, _ROT_B, _ROT_A, _ROT_B, _ROT_A)
    for i in range(5):
        for r in rots[i]:
            x0, x1 = rnd(x0, x1, r)
        x0 = x0 + ks[(i + 1) % 3]
        x1 = x1 + ks[(i + 2) % 3] + jnp.uint32(i + 1)
    return x0 ^ x1


def _make(B, V, K, NS, interpret=False):
    nchunk = (V + K - 1) // K
    nsub = K // 128

    def body(l_ref, out_ref, m_ref, s_ref, bval_ref, bidx_ref):
        j = pl.program_id(0)

        @pl.when(j == 0)
        def _init():
            m_ref[...] = jnp.full_like(m_ref, -jnp.inf)
            s_ref[...] = jnp.zeros_like(s_ref)
            bval_ref[...] = jnp.full_like(bval_ref, -jnp.inf)
            bidx_ref[...] = jnp.zeros_like(bidx_ref)

        lane = jax.lax.broadcasted_iota(jnp.int32, (B, 128), 1)
        rowbase = (jax.lax.broadcasted_iota(jnp.int32, (B, 128), 0) * V).astype(jnp.uint32)
        base_col = j * K

        def sub(t, _):
            col = lane + (base_col + t * 128)
            l = l_ref[:, pl.ds(t * 128, 128)]
            l = jnp.where(col < V, l, jnp.float32(-jnp.inf))
            m_ref[...] = jnp.maximum(m_ref[...], l)
            s_ref[...] = s_ref[...] + jnp.exp(l)
            off = rowbase + col.astype(jnp.uint32)
            for s in range(NS):
                bits = _threefry_bits(off + jnp.uint32(s * B * V))
                fb = (bits >> jnp.uint32(9)) | jnp.uint32(0x3F800000)
                fl = jax.lax.bitcast_convert_type(fb, jnp.float32) - jnp.float32(1.0)
                u = jnp.maximum(_TINY, fl * (jnp.float32(1.0) - _TINY) + _TINY)
                g = -jnp.log(-jnp.log(u))
                phi = g + l
                bv = bval_ref[pl.ds(s * B, B), :]
                better = phi > bv
                bval_ref[pl.ds(s * B, B), :] = jnp.where(better, phi, bv)
                bidx_ref[pl.ds(s * B, B), :] = jnp.where(
                    better, col, bidx_ref[pl.ds(s * B, B), :])
            return 0

        jax.lax.fori_loop(0, nsub, sub, 0, unroll=False)

        @pl.when(j == nchunk - 1)
        def _fin():
            m_row = jnp.max(m_ref[...], axis=1, keepdims=True)
            s_row = jnp.sum(s_ref[...], axis=1, keepdims=True)
            top_prob = jnp.exp(m_row) / s_row
            idxs = []
            for s in range(NS):
                bv = bval_ref[pl.ds(s * B, B), :]
                vmax = jnp.max(bv, axis=1, keepdims=True)
                cand = jnp.where(bv == vmax, bidx_ref[pl.ds(s * B, B), :],
                                 jnp.int32(0x7FFFFFFF))
                idxs.append(jnp.min(cand, axis=1, keepdims=True))
            agree_f = jnp.ones_like(top_prob)
            for t in range(1, NS):
                agree_f = agree_f + (idxs[t] == idxs[0]).astype(jnp.float32)
            agree_f = agree_f * jnp.float32(1.0 / NS)
            uniq = jnp.ones_like(top_prob)
            for t in range(1, NS):
                seen = idxs[t] == idxs[0]
                for t2 in range(1, t):
                    seen = jnp.logical_or(seen, idxs[t] == idxs[t2])
                uniq = uniq + jnp.float32(1.0) - seen.astype(jnp.float32)
            out_ref[...] = jnp.zeros_like(out_ref)
            out_ref[:, 0:1] = agree_f
            out_ref[:, 1:2] = uniq * jnp.float32(1.0 / NS)
            out_ref[:, 2:3] = agree_f - top_prob

    return pl.pallas_call(
        body,
        grid=(nchunk,),
        in_specs=[pl.BlockSpec((B, K), lambda j: (0, j))],
        out_specs=pl.BlockSpec((B, 128), lambda j: (0, 0)),
        out_shape=jax.ShapeDtypeStruct((B, 128), jnp.float32),
        scratch_shapes=[
            pltpu.VMEM((B, 128), jnp.float32),
            pltpu.VMEM((B, 128), jnp.float32),
            pltpu.VMEM((NS * B, 128), jnp.float32),
            pltpu.VMEM((NS * B, 128), jnp.int32),
        ],
        interpret=interpret,
    )


def kernel(logits):
    out = _make(_B, _V, _CHUNK, _NS)(logits)
    return out[:, :3]


# speedup vs baseline: 1.6722x; 1.0396x over previous
import numpy as np
import jax
import jax.numpy as jnp
from jax.experimental import pallas as pl
from jax.experimental.pallas import tpu as pltpu

_B = 32
_V = 1_000_000
_NS = 10
_CHUNK = 8192

_ROT_A = (13, 15, 26, 6)
_ROT_B = (17, 29, 16, 24)
_TINY = np.float32(np.finfo(np.float32).tiny)


def _np_threefry(k1, k2, x0, x1):
    ks = [np.uint32(k1), np.uint32(k2),
          np.uint32(k1) ^ np.uint32(k2) ^ np.uint32(0x1BD11BDA)]
    x0 = np.uint32(x0 + ks[0])
    x1 = np.uint32(x1 + ks[1])
    rots = (_ROT_A, _ROT_B, _ROT_A, _ROT_B, _ROT_A)
    for i in range(5):
        for r in rots[i]:
            x0 = np.uint32(x0 + x1)
            x1 = np.uint32(((x1 << np.uint32(r)) | (x1 >> np.uint32(32 - r))) & 0xFFFFFFFF)
            x1 = np.uint32(x1 ^ x0)
        x0 = np.uint32(x0 + ks[(i + 1) % 3])
        x1 = np.uint32(x1 + ks[(i + 2) % 3] + np.uint32(i + 1))
    return x0, x1


_K1, _K2 = _np_threefry(np.uint32(0), np.uint32(0), np.uint32(0), np.uint32(1))
_K3 = np.uint32(_K1 ^ _K2 ^ np.uint32(0x1BD11BDA))


def _threefry_bits(x1):
    ks = (jnp.uint32(_K1), jnp.uint32(_K2), jnp.uint32(_K3))
    x0 = jnp.full_like(x1, _K1)
    x1 = x1 + jnp.uint32(_K2)

    def rnd(a, b, r):
        a = a + b
        b = ((b << jnp.uint32(r)) | (b >> jnp.uint32(32 - r))) ^ a
        return a, b

    rots = (_ROT_A, _ROT_B, _ROT_A, _ROT_B, _ROT_A)
    for i in range(5):
        for r in rots[i]:
            x0, x1 = rnd(x0, x1, r)
        x0 = x0 + ks[(i + 1) % 3]
        x1 = x1 + ks[(i + 2) % 3] + jnp.uint32(i + 1)
    return x0 ^ x1


def _make(B, V, K, NS, interpret=False):
    nchunk = (V + K - 1) // K
    nsub = K // 128

    def body(l_ref, out_ref, m_ref, s_ref, bval_ref, bidx_ref):
        j = pl.program_id(0)

        @pl.when(j == 0)
        def _init():
            m_ref[...] = jnp.full_like(m_ref, -jnp.inf)
            s_ref[...] = jnp.zeros_like(s_ref)
            bval_ref[...] = jnp.full_like(bval_ref, -jnp.inf)
            bidx_ref[...] = jnp.zeros_like(bidx_ref)

        lane = jax.lax.broadcasted_iota(jnp.int32, (B, 128), 1)
        rowbase = (jax.lax.broadcasted_iota(jnp.int32, (B, 128), 0) * V).astype(jnp.uint32)
        base_col = j * K

        def sub(t, _):
            col = lane + (base_col + t * 128)
            l = l_ref[:, pl.ds(t * 128, 128)]
            l = jnp.where(col < V, l, jnp.float32(-jnp.inf))
            m_ref[...] = jnp.maximum(m_ref[...], l)
            s_ref[...] = s_ref[...] + jnp.exp(l)
            off = rowbase + col.astype(jnp.uint32)
            for s in range(NS):
                bits = _threefry_bits(off + jnp.uint32(s * B * V))
                fb = (bits >> jnp.uint32(9)) | jnp.uint32(0x3F800000)
                fl = jax.lax.bitcast_convert_type(fb, jnp.float32) - jnp.float32(1.0)
                u = jnp.maximum(_TINY, fl)
                g = -jnp.log(-jnp.log(u))
                phi = g + l
                bv = bval_ref[pl.ds(s * B, B), :]
                better = phi > bv
                bval_ref[pl.ds(s * B, B), :] = jnp.where(better, phi, bv)
                bidx_ref[pl.ds(s * B, B), :] = jnp.where(
                    better, col, bidx_ref[pl.ds(s * B, B), :])
            return 0

        jax.lax.fori_loop(0, nsub, sub, 0, unroll=2)

        @pl.when(j == nchunk - 1)
        def _fin():
            m_row = jnp.max(m_ref[...], axis=1, keepdims=True)
            s_row = jnp.sum(s_ref[...], axis=1, keepdims=True)
            top_prob = jnp.exp(m_row) / s_row
            idxs = []
            for s in range(NS):
                bv = bval_ref[pl.ds(s * B, B), :]
                vmax = jnp.max(bv, axis=1, keepdims=True)
                cand = jnp.where(bv == vmax, bidx_ref[pl.ds(s * B, B), :],
                                 jnp.int32(0x7FFFFFFF))
                idxs.append(jnp.min(cand, axis=1, keepdims=True))
            agree_f = jnp.ones_like(top_prob)
            for t in range(1, NS):
                agree_f = agree_f + (idxs[t] == idxs[0]).astype(jnp.float32)
            agree_f = agree_f * jnp.float32(1.0 / NS)
            uniq = jnp.ones_like(top_prob)
            for t in range(1, NS):
                seen = idxs[t] == idxs[0]
                for t2 in range(1, t):
                    seen = jnp.logical_or(seen, idxs[t] == idxs[t2])
                uniq = uniq + jnp.float32(1.0) - seen.astype(jnp.float32)
            out_ref[...] = jnp.zeros_like(out_ref)
            out_ref[:, 0:1] = agree_f
            out_ref[:, 1:2] = uniq * jnp.float32(1.0 / NS)
            out_ref[:, 2:3] = agree_f - top_prob

    return pl.pallas_call(
        body,
        grid=(nchunk,),
        in_specs=[pl.BlockSpec((B, K), lambda j: (0, j))],
        out_specs=pl.BlockSpec((B, 128), lambda j: (0, 0)),
        out_shape=jax.ShapeDtypeStruct((B, 128), jnp.float32),
        scratch_shapes=[
            pltpu.VMEM((B, 128), jnp.float32),
            pltpu.VMEM((B, 128), jnp.float32),
            pltpu.VMEM((NS * B, 128), jnp.float32),
            pltpu.VMEM((NS * B, 128), jnp.int32),
        ],
        interpret=interpret,
    )


def kernel(logits):
    out = _make(_B, _V, _CHUNK, _NS)(logits)
    return out[:, :3]
